# baseline (speedup 1.0000x reference)
import functools

import jax
import jax.numpy as jnp
from jax import lax
from jax.experimental import pallas as pl
from jax.experimental.pallas import tpu as pltpu
from jax.experimental.pallas import tpu_sc as plsc

EMBED_DIM = 64
NUM_CORES = 2
NUM_SUBCORES = 16
NUM_WORKERS = NUM_CORES * NUM_SUBCORES
BBLK = 128
NBUF = 4
TBUF = 2


def _sc_transpose_table(wt):
    D, V = wt.shape
    n_full = V // 128
    v_tail = V - n_full * 128
    mesh = plsc.VectorSubcoreMesh(core_axis_name="c", subcore_axis_name="s")

    @functools.partial(
        pl.kernel,
        mesh=mesh,
        out_type=jax.ShapeDtypeStruct((V // 2, 128), jnp.float32),
        scratch_types=[
            pltpu.VMEM((2, D, 128), jnp.float32),
            pltpu.VMEM((2, D, 130), jnp.float32),
            pltpu.VMEM((D, 64), jnp.float32),
            pltpu.SemaphoreType.DMA((2,)),
            pltpu.SemaphoreType.DMA((2,)),
        ],
        compiler_params=pltpu.CompilerParams(
            use_tc_tiling_on_sc=True, needs_layout_passes=False
        ),
    )
    def body(wt_hbm, out_hbm, ibuf, sbuf, tbuf, isem, osem):
        wid = lax.axis_index("s") * NUM_CORES + lax.axis_index("c")
        nk = 244 + jnp.where(wid < n_full - 244 * NUM_WORKERS, 1, 0)
        iota = lax.iota(jnp.int32, 16)
        q_m = [(iota + 16 * m) >> 1 for m in range(8)]
        j_m = [((iota + 16 * m) & 1) * 64 for m in range(8)]

        def fire_in(k, slot):
            col = wid + NUM_WORKERS * k
            pltpu.async_copy(
                wt_hbm.at[:, pl.ds(col * 128, 128)], ibuf.at[slot], isem.at[slot]
            )

        def wait_in(slot):
            pltpu.make_async_copy(
                wt_hbm.at[:, pl.ds(0, 128)], ibuf.at[slot], isem.at[slot]
            ).wait()

        def fire_out(k, slot):
            col = wid + NUM_WORKERS * k
            pltpu.async_copy(
                sbuf.at[slot, :, pl.ds(0, 128)],
                out_hbm.at[pl.ds(col * 64, 64)],
                osem.at[slot],
            )

        def wait_out(slot):
            pltpu.make_async_copy(
                sbuf.at[slot, :, pl.ds(0, 128)],
                out_hbm.at[pl.ds(0, 64)],
                osem.at[slot],
            ).wait()

        def transpose_col(src, slot, n_seg):
            dst = sbuf.at[slot]

            @plsc.parallel_loop(0, D, unroll=8)
            def _(d):
                dv = jnp.broadcast_to(d, (16,))
                for m in range(n_seg):
                    vals = src[d, pl.ds(16 * m, 16)]
                    plsc.store_scatter(dst, [q_m[m], j_m[m] + dv], vals)

        fire_in(0, 0)

        @pl.when(nk > 1)
        def _():
            fire_in(1, 1)

        def step(k, carry):
            s = k % 2
            wait_in(s)

            @pl.when(k >= 2)
            def _():
                wait_out(s)

            transpose_col(ibuf.at[s], s, 8)
            fire_out(k, s)

            @pl.when(k + 2 < nk)
            def _():
                fire_in(k + 2, s)

            return carry

        lax.fori_loop(0, nk, step, 0)
        for s in range(2):
            @pl.when(nk > s)
            def _():
                wait_out(s)

        if v_tail:
            @pl.when(wid == n_full % NUM_WORKERS)
            def _():
                pltpu.sync_copy(wt_hbm.at[:, pl.ds(n_full * 128, v_tail)], tbuf)
                transpose_col(tbuf, 0, v_tail // 16)
                pltpu.sync_copy(
                    sbuf.at[0, pl.ds(0, v_tail // 2), pl.ds(0, 128)],
                    out_hbm.at[pl.ds(V // 2 - v_tail // 2, v_tail // 2)],
                )

    return body(wt)


@functools.partial(jax.jit, static_argnames=("n_l",))
def _sc_embedding_lookup(ids, weight, *, n_l):
    lt_n = n_l // 8
    mesh = plsc.VectorSubcoreMesh(core_axis_name="c", subcore_axis_name="s")

    @functools.partial(
        pl.kernel,
        mesh=mesh,
        out_type=jax.ShapeDtypeStruct(
            (n_l, EMBED_DIM // 8, NUM_WORKERS, 8, BBLK), jnp.float32
        ),
        scratch_types=[
            pltpu.VMEM((lt_n, 8, BBLK), jnp.int32),
            pltpu.VMEM((NBUF, BBLK, EMBED_DIM), jnp.float32),
            pltpu.VMEM((TBUF, EMBED_DIM // 8, 8, BBLK + 1), jnp.float32),
            pltpu.SemaphoreType.DMA((NBUF,)),
            pltpu.SemaphoreType.DMA((TBUF,)),
        ],
        compiler_params=pltpu.CompilerParams(
            use_tc_tiling_on_sc=False, needs_layout_passes=False
        ),
    )
    def body(ids_hbm, w_hbm, out_hbm, idx_v, rows_v, tps_v, gsem, tsem):
        wid = lax.axis_index("s") * NUM_CORES + lax.axis_index("c")

        pltpu.sync_copy(ids_hbm.at[:, wid], idx_v)

        def fire_gather(j, slot):
            row = idx_v.at[j // 8, j % 8]
            pltpu.async_copy(w_hbm.at[row], rows_v.at[slot], gsem.at[slot])

        def wait_gather(slot):
            pltpu.make_async_copy(
                w_hbm.at[idx_v.at[0, 0]], rows_v.at[slot], gsem.at[slot]
            ).wait()

        def fire_store(l, t):
            pltpu.async_copy(
                tps_v.at[t, :, :, pl.ds(0, BBLK)], out_hbm.at[l, :, wid], tsem.at[t]
            )

        def wait_store(t):
            pltpu.make_async_copy(
                tps_v.at[t, :, :, pl.ds(0, BBLK)], out_hbm.at[0, :, 0], tsem.at[t]
            ).wait()

        iota = lax.iota(jnp.int32, 16)
        seg_d = [iota + 16 * k for k in range(EMBED_DIM // 16)]
        idx_dt = [sd // 8 for sd in seg_d]
        idx_ds = [sd % 8 for sd in seg_d]

        def transpose_block(s, t):
            src = rows_v.at[s]
            dst = tps_v.at[t]

            @plsc.parallel_loop(0, BBLK, unroll=8)
            def _(bl):
                col = jnp.broadcast_to(bl, (16,))
                row = src.at[bl]
                for k in range(EMBED_DIM // 16):
                    vals = row[pl.ds(16 * k, 16)]
                    plsc.store_scatter(dst, [idx_dt[k], idx_ds[k], col], vals)

        for j in range(NBUF - 1):
            fire_gather(j, j)

        def step(l, carry):
            s = l % NBUF
            t = l % TBUF

            @pl.when(l < n_l - (NBUF - 1))
            def _():
                fire_gather(l + NBUF - 1, (l + NBUF - 1) % NBUF)

            wait_gather(s)

            @pl.when(l >= TBUF)
            def _():
                wait_store(t)

            transpose_block(s, t)
            fire_store(l, t)
            return carry

        lax.fori_loop(0, n_l, step, 0)
        for t in range(TBUF):
            wait_store(t)

    return body(ids, weight)


def kernel(token_ids, weight):
    B, L = token_ids.shape
    assert B // BBLK == NUM_WORKERS and B % BBLK == 0
    assert L % 8 == 0 and EMBED_DIM % 8 == 0
    ids_view = (
        token_ids.astype(jnp.int32)
        .T.reshape(L // 8, 8, B // BBLK, BBLK)
        .transpose(0, 2, 1, 3)
    )
    wp = _sc_transpose_table(weight.T)
    out5 = _sc_embedding_lookup(ids_view, wp.reshape(-1, EMBED_DIM), n_l=L)
    return out5.transpose(2, 4, 0, 1, 3).reshape(B, L, EMBED_DIM)


# speedup vs baseline: 1.3624x; 1.3624x over previous
import functools

import jax
import jax.numpy as jnp
from jax import lax
from jax.experimental import pallas as pl
from jax.experimental.pallas import tpu as pltpu
from jax.experimental.pallas import tpu_sc as plsc

EMBED_DIM = 64
NUM_CORES = 2
NUM_SUBCORES = 16
NUM_WORKERS = NUM_CORES * NUM_SUBCORES
BBLK = 128
NBUF = 4
TBUF = 2


@functools.partial(jax.jit, static_argnames=("n_l",))
def _sc_embedding_lookup(ids, weight, *, n_l):
    lt_n = n_l // 8
    mesh = plsc.VectorSubcoreMesh(core_axis_name="c", subcore_axis_name="s")

    @functools.partial(
        pl.kernel,
        mesh=mesh,
        out_type=jax.ShapeDtypeStruct(
            (n_l, EMBED_DIM // 8, NUM_WORKERS, 8, BBLK), jnp.float32
        ),
        scratch_types=[
            pltpu.VMEM((lt_n, 8, BBLK), jnp.int32),
            pltpu.VMEM((NBUF, BBLK, EMBED_DIM), jnp.float32),
            pltpu.VMEM((TBUF, EMBED_DIM // 8, 8, BBLK + 1), jnp.float32),
            pltpu.SemaphoreType.DMA((NBUF,)),
            pltpu.SemaphoreType.DMA((TBUF,)),
        ],
        compiler_params=pltpu.CompilerParams(
            use_tc_tiling_on_sc=False, needs_layout_passes=False
        ),
    )
    def body(ids_hbm, w_hbm, out_hbm, idx_v, rows_v, tps_v, gsem, tsem):
        wid = lax.axis_index("s") * NUM_CORES + lax.axis_index("c")

        pltpu.sync_copy(ids_hbm.at[:, wid], idx_v)

        def fire_gather(j, slot):
            row = idx_v.at[j // 8, j % 8]
            pltpu.async_copy(w_hbm.at[row], rows_v.at[slot], gsem.at[slot])

        def wait_gather(slot):
            pltpu.make_async_copy(
                w_hbm.at[idx_v.at[0, 0]], rows_v.at[slot], gsem.at[slot]
            ).wait()

        def fire_store(l, t):
            pltpu.async_copy(
                tps_v.at[t, :, :, pl.ds(0, BBLK)], out_hbm.at[l, :, wid], tsem.at[t]
            )

        def wait_store(t):
            pltpu.make_async_copy(
                tps_v.at[t, :, :, pl.ds(0, BBLK)], out_hbm.at[0, :, 0], tsem.at[t]
            ).wait()

        iota = lax.iota(jnp.int32, 16)
        seg_d = [iota + 16 * k for k in range(EMBED_DIM // 16)]
        idx_dt = [sd // 8 for sd in seg_d]
        idx_ds = [sd % 8 for sd in seg_d]

        def transpose_block(s, t):
            src = rows_v.at[s]
            dst = tps_v.at[t]

            @plsc.parallel_loop(0, BBLK, unroll=8)
            def _(bl):
                col = jnp.broadcast_to(bl, (16,))
                row = src.at[bl]
                for k in range(EMBED_DIM // 16):
                    vals = row[pl.ds(16 * k, 16)]
                    plsc.store_scatter(dst, [idx_dt[k], idx_ds[k], col], vals)

        for j in range(NBUF - 1):
            fire_gather(j, j)

        def step(l, carry):
            s = l % NBUF
            t = l % TBUF

            @pl.when(l < n_l - (NBUF - 1))
            def _():
                fire_gather(l + NBUF - 1, (l + NBUF - 1) % NBUF)

            wait_gather(s)

            @pl.when(l >= TBUF)
            def _():
                wait_store(t)

            transpose_block(s, t)
            fire_store(l, t)
            return carry

        lax.fori_loop(0, n_l, step, 0)
        for t in range(TBUF):
            wait_store(t)

    return body(ids, weight)


def kernel(token_ids, weight):
    B, L = token_ids.shape
    assert B // BBLK == NUM_WORKERS and B % BBLK == 0
    assert L % 8 == 0 and EMBED_DIM % 8 == 0
    ids_view = (
        token_ids.astype(jnp.int32)
        .T.reshape(L // 8, 8, B // BBLK, BBLK)
        .transpose(0, 2, 1, 3)
    )
    out5 = _sc_embedding_lookup(ids_view, weight, n_l=L)
    return out5.transpose(2, 4, 0, 1, 3).reshape(B, L, EMBED_DIM)
